# baseline (speedup 1.0000x reference)
import functools

import jax
import jax.numpy as jnp
from jax import lax
from jax.experimental import pallas as pl
from jax.experimental.pallas import tpu as pltpu
from jax.experimental.pallas import tpu_sc as plsc

_NUM_ENTITIES = 1000000
_NUM_RELATIONS = 1000
_D = 64
_B = 16384
_NW = 32
_BPW = _B // _NW
_L = 16


def _make_kernel():
    mesh = plsc.VectorSubcoreMesh(core_axis_name="c", subcore_axis_name="s")

    @functools.partial(
        pl.kernel,
        out_type=jax.ShapeDtypeStruct((_B,), jnp.float32),
        mesh=mesh,
        compiler_params=pltpu.CompilerParams(
            needs_layout_passes=False, use_tc_tiling_on_sc=False),
        scratch_types=[
            pltpu.VMEM((_BPW,), jnp.int32),
            pltpu.VMEM((_BPW,), jnp.int32),
            pltpu.VMEM((_BPW,), jnp.int32),
            pltpu.VMEM((_BPW, _D), jnp.float32),
            pltpu.VMEM((_BPW, _D), jnp.float32),
            pltpu.VMEM((_BPW, _D), jnp.float32),
            pltpu.VMEM((_BPW,), jnp.float32),
            pltpu.VMEM((_L * _BPW,), jnp.float32),
            pltpu.SemaphoreType.DMA,
        ],
    )
    def k(ent_hbm, heads_hbm, rels_hbm, tails_hbm, relt_hbm, out_hbm,
          hidx_v, ridx_v, tidx_v, h_v, r_v, t_v, out_v, part_v, sem):
        wid = lax.axis_index("s") * 2 + lax.axis_index("c")
        base = wid * _BPW
        pltpu.sync_copy(heads_hbm.at[pl.ds(base, _BPW)], hidx_v)
        pltpu.sync_copy(rels_hbm.at[pl.ds(base, _BPW)], ridx_v)
        pltpu.sync_copy(tails_hbm.at[pl.ds(base, _BPW)], tidx_v)
        ch = pltpu.async_copy(ent_hbm.at[hidx_v], h_v, sem)
        cr = pltpu.async_copy(relt_hbm.at[ridx_v], r_v, sem)
        ct = pltpu.async_copy(ent_hbm.at[tidx_v], t_v, sem)
        ch.wait()
        cr.wait()
        ct.wait()

        scat = lax.iota(jnp.int32, _L) * _BPW

        def row(i, _):
            acc = jnp.zeros((_L,), jnp.float32)
            for j in range(_D // _L):
                sl = pl.ds(j * _L, _L)
                acc = acc + h_v[i, sl] * r_v[i, sl] * t_v[i, sl]
            plsc.store_scatter(part_v, [scat + i], acc)
            return 0

        lax.fori_loop(0, _BPW, row, 0)

        def group(g, _):
            acc = part_v[pl.ds(g * _L, _L)]
            for l in range(1, _L):
                acc = acc + part_v[pl.ds(l * _BPW + g * _L, _L)]
            out_v[pl.ds(g * _L, _L)] = acc
            return 0

        lax.fori_loop(0, _BPW // _L, group, 0)
        pltpu.sync_copy(out_v, out_hbm.at[pl.ds(base, _BPW)])

    return k


_kernel_call = _make_kernel()


def kernel(entity_emb, heads, relations, tails, rel_table):
    return _kernel_call(
        entity_emb,
        heads.astype(jnp.int32),
        relations.astype(jnp.int32),
        tails.astype(jnp.int32),
        rel_table,
    )


# speedup vs baseline: 1.6717x; 1.6717x over previous
import functools

import jax
import jax.numpy as jnp
from jax import lax
from jax.experimental import pallas as pl
from jax.experimental.pallas import tpu as pltpu
from jax.experimental.pallas import tpu_sc as plsc

_NUM_ENTITIES = 1000000
_NUM_RELATIONS = 1000
_D = 64
_B = 16384
_NW = 32
_BPW = _B // _NW
_C = 256
_NCHUNK = _BPW // _C
_L = 16


def _make_kernel():
    mesh = plsc.VectorSubcoreMesh(core_axis_name="c", subcore_axis_name="s")

    @functools.partial(
        pl.kernel,
        out_type=jax.ShapeDtypeStruct((_B,), jnp.float32),
        mesh=mesh,
        compiler_params=pltpu.CompilerParams(
            needs_layout_passes=False, use_tc_tiling_on_sc=True),
        scratch_types=[
            pltpu.VMEM((_BPW,), jnp.int32),
            pltpu.VMEM((_BPW,), jnp.int32),
            pltpu.VMEM((_BPW,), jnp.int32),
            pltpu.VMEM((_C, _D), jnp.float32),
            pltpu.VMEM((_C, _D), jnp.float32),
            pltpu.VMEM((_C, _D), jnp.float32),
            pltpu.VMEM((_BPW,), jnp.float32),
            pltpu.VMEM((_L * _C,), jnp.float32),
            pltpu.SemaphoreType.DMA,
            pltpu.SemaphoreType.DMA,
        ],
    )
    def k(ent_hbm, heads_hbm, rels_hbm, tails_hbm, relt_hbm, out_hbm,
          hidx_v, ridx_v, tidx_v, h_v, r_v, t_v, out_v, part_v, sem, gsem):
        wid = lax.axis_index("s") * 2 + lax.axis_index("c")
        base = wid * _BPW
        pltpu.sync_copy(heads_hbm.at[pl.ds(base, _BPW)], hidx_v)
        pltpu.sync_copy(rels_hbm.at[pl.ds(base, _BPW)], ridx_v)
        pltpu.sync_copy(tails_hbm.at[pl.ds(base, _BPW)], tidx_v)

        scat = lax.iota(jnp.int32, _L) * _C

        def chunk(c, _):
            c0 = c * _C

            def fire(g, _):
                hvec = hidx_v[pl.ds(c0 + g * _L, _L)]
                rvec = ridx_v[pl.ds(c0 + g * _L, _L)]
                tvec = tidx_v[pl.ds(c0 + g * _L, _L)]
                for j in range(_L):
                    i = g * _L + j
                    pltpu.async_copy(ent_hbm.at[pl.ds(hvec[j], 1)],
                                     h_v.at[pl.ds(i, 1)], gsem)
                    pltpu.async_copy(relt_hbm.at[pl.ds(rvec[j], 1)],
                                     r_v.at[pl.ds(i, 1)], gsem)
                    pltpu.async_copy(ent_hbm.at[pl.ds(tvec[j], 1)],
                                     t_v.at[pl.ds(i, 1)], gsem)
                return 0

            lax.fori_loop(0, _C // _L, fire, 0)

            def drain(i, _):
                pltpu.make_async_copy(ent_hbm.at[pl.ds(0, 1)],
                                      h_v.at[pl.ds(i, 1)], gsem).wait()
                pltpu.make_async_copy(relt_hbm.at[pl.ds(0, 1)],
                                      r_v.at[pl.ds(i, 1)], gsem).wait()
                pltpu.make_async_copy(ent_hbm.at[pl.ds(0, 1)],
                                      t_v.at[pl.ds(i, 1)], gsem).wait()
                return 0

            lax.fori_loop(0, _C, drain, 0)

            def row(i, _):
                acc = jnp.zeros((_L,), jnp.float32)
                for j in range(_D // _L):
                    sl = pl.ds(j * _L, _L)
                    acc = acc + h_v[i, sl] * r_v[i, sl] * t_v[i, sl]
                plsc.store_scatter(part_v, [scat + i], acc)
                return 0

            lax.fori_loop(0, _C, row, 0)

            def group(g, _):
                acc = part_v[pl.ds(g * _L, _L)]
                for l in range(1, _L):
                    acc = acc + part_v[pl.ds(l * _C + g * _L, _L)]
                out_v[pl.ds(c0 + g * _L, _L)] = acc
                return 0

            lax.fori_loop(0, _C // _L, group, 0)
            return 0

        lax.fori_loop(0, _NCHUNK, chunk, 0)
        pltpu.sync_copy(out_v, out_hbm.at[pl.ds(base, _BPW)])

    return k


_kernel_call = _make_kernel()


def kernel(entity_emb, heads, relations, tails, rel_table):
    return _kernel_call(
        entity_emb,
        heads.astype(jnp.int32),
        relations.astype(jnp.int32),
        tails.astype(jnp.int32),
        rel_table,
    )
